# Initial kernel scaffold; baseline (speedup 1.0000x reference)
import jax
import jax.numpy as jnp
from jax.experimental import pallas as pl


def kernel(nodes, start_index, end_index, W_ne, b_ne, W_ee, b_ee, W_nn, b_nn, W_en, b_en, W_pe, b_pe):
    raise NotImplementedError("write your pallas kernel here")


# speedup vs baseline: 2.4613x; 2.4613x over previous
import functools

import jax
import jax.numpy as jnp
from jax import lax
from jax.experimental import pallas as pl
from jax.experimental.pallas import tpu as pltpu
from jax.experimental.pallas import tpu_sc as plsc

N = 10000
E = 320000
H = 128

NC = 2
NS = 16
NW = NC * NS
EPW = E // NW
K = 80
NCHUNK = EPW // K
NP = 10240
RPS = NP // NS

KQ = 2000
NQCH = EPW // KQ

_mesh = plsc.VectorSubcoreMesh(core_axis_name="c", subcore_axis_name="s")


def _zero_rows(buf, rows):
    zv = jnp.zeros((16,), jnp.float32)

    @pl.loop(0, rows)
    def _(r):
        for c in range(H // 16):
            buf[r, pl.ds(16 * c, 16)] = zv


def _acc_init(acc, zbuf, s):
    _zero_rows(zbuf, K)
    base = s * RPS
    for j in range(RPS // K):
        pltpu.sync_copy(zbuf, acc.at[pl.ds(base + j * K, K)])
    rem = RPS % K
    if rem:
        pltpu.sync_copy(zbuf.at[pl.ds(0, rem)],
                        acc.at[pl.ds(base + (RPS // K) * K, rem)])


def _acc_writeout(acc, msg_hbm, c, s):
    base = s * RPS
    for j in range(RPS // K):
        pltpu.sync_copy(acc.at[pl.ds(base + j * K, K)],
                        msg_hbm.at[c, pl.ds(base + j * K, K)])
    rem = RPS % K
    if rem:
        pltpu.sync_copy(acc.at[pl.ds(base + (RPS // K) * K, rem)],
                        msg_hbm.at[c, pl.ds(base + (RPS // K) * K, rem)])


@functools.partial(
    pl.kernel,
    mesh=_mesh,
    out_type=[
        jax.ShapeDtypeStruct((E, H), jnp.float32),
        jax.ShapeDtypeStruct((E, H), jnp.float32),
        jax.ShapeDtypeStruct((NC, NP, H), jnp.float32),
    ],
    scratch_types=[
        pltpu.VMEM((K,), jnp.int32),
        pltpu.VMEM((K,), jnp.int32),
        pltpu.VMEM((K, H), jnp.float32),
        pltpu.VMEM((K, H), jnp.float32),
        pltpu.VMEM((K, H), jnp.float32),
        pltpu.VMEM((K, H), jnp.float32),
        pltpu.VMEM((H,), jnp.float32),
        pltpu.VMEM_SHARED((NP, H), jnp.float32),
        pltpu.SemaphoreType.DMA,
        pltpu.SemaphoreType.DMA,
        pltpu.SemaphoreType.DMA,
        pltpu.SemaphoreType.DMA,
    ],
)
def _sc_pass1(ns_h, nd_h, a_h, b_h, src_h, dst_h, bee_h,
              e1_h, z2_h, msg_h,
              idx_s, idx_d, gs, gd, ga, gb, bias_v, acc,
              sem0, sem1, sem2, sem3):
    c = lax.axis_index("c")
    s = lax.axis_index("s")
    wid = s * NC + c
    base = wid * EPW

    pltpu.sync_copy(bee_h, bias_v)
    _acc_init(acc, gs, s)
    plsc.subcore_barrier()

    @pl.loop(0, NCHUNK)
    def _(i):
        b0 = base + i * K
        pltpu.sync_copy(src_h.at[pl.ds(b0, K)], idx_s)
        pltpu.sync_copy(dst_h.at[pl.ds(b0, K)], idx_d)
        cp0 = pltpu.async_copy(ns_h.at[idx_s], gs, sem0)
        cp1 = pltpu.async_copy(nd_h.at[idx_d], gd, sem1)
        cp2 = pltpu.async_copy(a_h.at[idx_s], ga, sem2)
        cp3 = pltpu.async_copy(b_h.at[idx_d], gb, sem3)
        cp0.wait()
        cp1.wait()
        cp2.wait()
        cp3.wait()

        @pl.loop(0, K)
        def _(r):
            for cc in range(H // 16):
                o = 16 * cc
                bv = bias_v[pl.ds(o, 16)]
                vs = gs[r, pl.ds(o, 16)]
                vd = gd[r, pl.ds(o, 16)]
                gs[r, pl.ds(o, 16)] = jnp.maximum(vs + vd + bv, 0.0)
                va = ga[r, pl.ds(o, 16)]
                vb = gb[r, pl.ds(o, 16)]
                ga[r, pl.ds(o, 16)] = va + vb

        pltpu.sync_copy(gs, e1_h.at[pl.ds(b0, K)])
        pltpu.sync_copy(ga, z2_h.at[pl.ds(b0, K)])
        pltpu.sync_copy(gs, acc.at[idx_d], add=True)

    plsc.subcore_barrier()
    _acc_writeout(acc, msg_h, c, s)


@functools.partial(
    pl.kernel,
    mesh=_mesh,
    out_type=[
        jax.ShapeDtypeStruct((E, H), jnp.float32),
        jax.ShapeDtypeStruct((NC, NP, H), jnp.float32),
    ],
    scratch_types=[
        pltpu.VMEM((K,), jnp.int32),
        pltpu.VMEM((K,), jnp.int32),
        pltpu.VMEM((K, H), jnp.float32),
        pltpu.VMEM((K, H), jnp.float32),
        pltpu.VMEM((K, H), jnp.float32),
        pltpu.VMEM_SHARED((NP, H), jnp.float32),
        pltpu.SemaphoreType.DMA,
        pltpu.SemaphoreType.DMA,
        pltpu.SemaphoreType.DMA,
    ],
)
def _sc_pass2(a_h, b_h, src_h, dst_h, e2_h,
              z3_h, msg_h,
              idx_s, idx_d, ga, gb, ge, acc,
              sem0, sem1, sem2):
    c = lax.axis_index("c")
    s = lax.axis_index("s")
    wid = s * NC + c
    base = wid * EPW

    _acc_init(acc, ga, s)
    plsc.subcore_barrier()

    @pl.loop(0, NCHUNK)
    def _(i):
        b0 = base + i * K
        pltpu.sync_copy(src_h.at[pl.ds(b0, K)], idx_s)
        pltpu.sync_copy(dst_h.at[pl.ds(b0, K)], idx_d)
        cp0 = pltpu.async_copy(a_h.at[idx_s], ga, sem0)
        cp1 = pltpu.async_copy(b_h.at[idx_d], gb, sem1)
        cp2 = pltpu.async_copy(e2_h.at[pl.ds(b0, K)], ge, sem2)
        cp0.wait()
        cp1.wait()
        cp2.wait()

        @pl.loop(0, K)
        def _(r):
            for cc in range(H // 16):
                o = 16 * cc
                va = ga[r, pl.ds(o, 16)]
                vb = gb[r, pl.ds(o, 16)]
                ga[r, pl.ds(o, 16)] = va + vb

        pltpu.sync_copy(ga, z3_h.at[pl.ds(b0, K)])
        pltpu.sync_copy(ge, acc.at[idx_d], add=True)

    plsc.subcore_barrier()
    _acc_writeout(acc, msg_h, c, s)


KQ = 2000
NQCH = EPW // KQ


@functools.partial(
    pl.kernel,
    mesh=_mesh,
    out_type=jax.ShapeDtypeStruct((E,), jnp.float32),
    compiler_params=pltpu.CompilerParams(needs_layout_passes=False),
    scratch_types=[
        pltpu.VMEM((N,), jnp.float32),
        pltpu.VMEM((N,), jnp.float32),
        pltpu.VMEM((KQ,), jnp.int32),
        pltpu.VMEM((KQ,), jnp.int32),
        pltpu.VMEM((KQ,), jnp.float32),
    ],
)
def _sc_passq(ps_h, pd_h, src_h, dst_h,
              q_h,
              pst, pdt, idx_s, idx_d, qb):
    c = lax.axis_index("c")
    s = lax.axis_index("s")
    wid = s * NC + c
    base = wid * EPW

    pltpu.sync_copy(ps_h, pst)
    pltpu.sync_copy(pd_h, pdt)

    @pl.loop(0, NQCH)
    def _(i):
        b0 = base + i * KQ
        pltpu.sync_copy(src_h.at[pl.ds(b0, KQ)], idx_s)
        pltpu.sync_copy(dst_h.at[pl.ds(b0, KQ)], idx_d)

        @pl.loop(0, KQ // 16)
        def _(j):
            vs = idx_s[pl.ds(16 * j, 16)]
            vd = idx_d[pl.ds(16 * j, 16)]
            va = plsc.load_gather(pst, [vs])
            vb = plsc.load_gather(pdt, [vd])
            qb[pl.ds(16 * j, 16)] = va + vb

        pltpu.sync_copy(qb, q_h.at[pl.ds(b0, KQ)])


def _tc_node_pre(nodes, W_ne, b_ne, W_ee, W_en):
    def body(nodes_ref, wne_ref, bne_ref, wee_ref, wen_ref,
             n0_ref, ns_ref, nd_ref, a_ref, b_ref):
        n0 = jnp.maximum(
            jnp.dot(nodes_ref[...], wne_ref[...],
                    preferred_element_type=jnp.float32) + bne_ref[...], 0.0)
        n0_ref[...] = n0
        ns_ref[...] = jnp.dot(n0, wee_ref[0:H, :],
                              preferred_element_type=jnp.float32)
        nd_ref[...] = jnp.dot(n0, wee_ref[H:2 * H, :],
                              preferred_element_type=jnp.float32)
        a_ref[...] = jnp.dot(n0, wen_ref[0:H, :],
                             preferred_element_type=jnp.float32)
        b_ref[...] = jnp.dot(n0, wen_ref[H:2 * H, :],
                             preferred_element_type=jnp.float32)

    shp = jax.ShapeDtypeStruct((N, H), jnp.float32)
    return pl.pallas_call(
        body,
        out_shape=[shp, shp, shp, shp, shp],
    )(nodes, W_ne, b_ne.reshape(1, H), W_ee, W_en)


def _tc_node_update(n, msgP, W_nn, b_nn, W_en):
    def body(n_ref, msg_ref, wnn_ref, bnn_ref, wen_ref,
             n1_ref, a_ref, b_ref):
        msg = msg_ref[0, 0:N, :] + msg_ref[1, 0:N, :]
        h = jnp.maximum(
            jnp.dot(n_ref[...], wnn_ref[0:H, :],
                    preferred_element_type=jnp.float32)
            + jnp.dot(msg, wnn_ref[H:2 * H, :],
                      preferred_element_type=jnp.float32)
            + bnn_ref[...], 0.0) + n_ref[...]
        n1_ref[...] = h
        a_ref[...] = jnp.dot(h, wen_ref[0:H, :],
                             preferred_element_type=jnp.float32)
        b_ref[...] = jnp.dot(h, wen_ref[H:2 * H, :],
                             preferred_element_type=jnp.float32)

    shp = jax.ShapeDtypeStruct((N, H), jnp.float32)
    return pl.pallas_call(
        body,
        out_shape=[shp, shp, shp],
    )(n, msgP, W_nn, b_nn.reshape(1, H), W_en)


def _tc_node_update2(n, msgP, W_nn, b_nn, Wpe_sd):
    def body(n_ref, msg_ref, wnn_ref, bnn_ref, wpe_ref, pspd_ref):
        msg = msg_ref[0, 0:N, :] + msg_ref[1, 0:N, :]
        h = jnp.maximum(
            jnp.dot(n_ref[...], wnn_ref[0:H, :],
                    preferred_element_type=jnp.float32)
            + jnp.dot(msg, wnn_ref[H:2 * H, :],
                      preferred_element_type=jnp.float32)
            + bnn_ref[...], 0.0) + n_ref[...]
        pspd_ref[...] = jnp.dot(h, wpe_ref[...],
                                preferred_element_type=jnp.float32)

    return pl.pallas_call(
        body,
        out_shape=jax.ShapeDtypeStruct((N, 8), jnp.float32),
    )(n, msgP, W_nn, b_nn.reshape(1, H), Wpe_sd)


BR = 2000


def _tc_edge_combine(e, z, W, b):
    def body(e_ref, z_ref, w_ref, b_ref, o_ref):
        o_ref[...] = jnp.maximum(
            z_ref[...] + jnp.dot(e_ref[...], w_ref[...],
                                 preferred_element_type=jnp.float32)
            + b_ref[...], 0.0) + e_ref[...]

    return pl.pallas_call(
        body,
        grid=(E // BR,),
        in_specs=[
            pl.BlockSpec((BR, H), lambda i: (i, 0)),
            pl.BlockSpec((BR, H), lambda i: (i, 0)),
            pl.BlockSpec((H, H), lambda i: (0, 0)),
            pl.BlockSpec((1, H), lambda i: (0, 0)),
        ],
        out_specs=pl.BlockSpec((BR, H), lambda i: (i, 0)),
        out_shape=jax.ShapeDtypeStruct((E, H), jnp.float32),
    )(e, z, W, b.reshape(1, H))


def _tc_final(e, z, q, W, b, wpe, bpe):
    def body(e_ref, z_ref, q_ref, w_ref, b_ref, wpe_ref, bpe_ref, o_ref):
        e3 = jnp.maximum(
            z_ref[...] + jnp.dot(e_ref[...], w_ref[...],
                                 preferred_element_type=jnp.float32)
            + b_ref[...], 0.0) + e_ref[...]
        o_ref[...] = (jnp.sum(e3 * wpe_ref[...], axis=1, keepdims=True)
                      + q_ref[...] + bpe_ref[...])

    return pl.pallas_call(
        body,
        grid=(E // BR,),
        in_specs=[
            pl.BlockSpec((BR, H), lambda i: (i, 0)),
            pl.BlockSpec((BR, H), lambda i: (i, 0)),
            pl.BlockSpec((BR, 1), lambda i: (i, 0)),
            pl.BlockSpec((H, H), lambda i: (0, 0)),
            pl.BlockSpec((1, H), lambda i: (0, 0)),
            pl.BlockSpec((1, H), lambda i: (0, 0)),
            pl.BlockSpec((1, 1), lambda i: (0, 0)),
        ],
        out_specs=pl.BlockSpec((BR, 1), lambda i: (i, 0)),
        out_shape=jax.ShapeDtypeStruct((E, 1), jnp.float32),
    )(e, z, q.reshape(E, 1), W, b.reshape(1, H), wpe.reshape(1, H),
      bpe.reshape(1, 1))


def kernel(nodes, start_index, end_index, W_ne, b_ne, W_ee, b_ee,
           W_nn, b_nn, W_en, b_en, W_pe, b_pe):
    src = start_index.astype(jnp.int32)
    dst = end_index.astype(jnp.int32)

    n0, ns, nd, A0, B0 = _tc_node_pre(nodes, W_ne, b_ne, W_ee, W_en)
    e1, z2, msg1 = _sc_pass1(ns, nd, A0, B0, src, dst, b_ee)
    n1, A1, B1 = _tc_node_update(n0, msg1, W_nn, b_nn, W_en)
    e2 = _tc_edge_combine(e1, z2, W_en[2 * H:], b_en)
    z3, msg2 = _sc_pass2(A1, B1, src, dst, e2)
    Wpe_sd = jnp.concatenate(
        [W_pe[0:H], W_pe[H:2 * H], jnp.zeros((H, 6), jnp.float32)], axis=1)
    pspd = _tc_node_update2(n1, msg2, W_nn, b_nn, Wpe_sd)
    q = _sc_passq(pspd[:, 0], pspd[:, 1], src, dst)
    out = _tc_final(e2, z3, q, W_en[2 * H:], b_en, W_pe[2 * H:, 0], b_pe[0])
    return out[:, 0]
